# baseline (speedup 1.0000x reference)
import functools

import jax
import jax.numpy as jnp
from jax.experimental import pallas as pl
from jax.experimental.pallas import tpu as pltpu


def _moe_kernel(x_hbm, rw_hbm, w1_hbm, b1_hbm, w2_hbm, b2_hbm, o_ref,
                xs, rws, b1s, b2s, w1s, w2s, hs, sems, sem1, sem2,
                *, n_experts):
    cp_small = [
        pltpu.make_async_copy(x_hbm, xs, sems.at[0]),
        pltpu.make_async_copy(rw_hbm, rws, sems.at[1]),
        pltpu.make_async_copy(b1_hbm, b1s, sems.at[2]),
        pltpu.make_async_copy(b2_hbm, b2s, sems.at[3]),
    ]
    for cp in cp_small:
        cp.start()
    for e in range(n_experts):
        pltpu.make_async_copy(w1_hbm.at[e], w1s.at[e], sem1.at[e]).start()
        pltpu.make_async_copy(w2_hbm.at[e], w2s.at[e], sem2.at[e]).start()
    for cp in cp_small:
        cp.wait()

    x = xs[...]
    xb = x.astype(jnp.bfloat16)

    logits = jax.lax.dot_general(
        x, rws[...], (((1,), (1,)), ((), ())),
        preferred_element_type=jnp.float32)
    m = jnp.max(logits, axis=1, keepdims=True)
    ex = jnp.exp(logits - m)
    probs = ex / jnp.sum(ex, axis=1, keepdims=True)

    L = probs.shape[0]
    col = jax.lax.broadcasted_iota(jnp.int32, (L, n_experts), 1)
    m1 = jnp.max(probs, axis=1, keepdims=True)
    eq1 = probs >= m1
    i1 = jnp.min(jnp.where(eq1, col, n_experts), axis=1, keepdims=True)
    mask1 = col == i1
    rest = jnp.where(mask1, -jnp.inf, probs)
    m2 = jnp.max(rest, axis=1, keepdims=True)
    eq2 = rest >= m2
    i2 = jnp.min(jnp.where(eq2, col, n_experts), axis=1, keepdims=True)
    mask2 = col == i2

    denom = m1 + m2 + 1e-9
    gates = (jnp.where(mask1, m1, 0.0) +
             jnp.where(mask2, m2, 0.0)) / denom

    o_ref[...] = jax.lax.dot_general(
        gates, b2s[...], (((1,), (0,)), ((), ())),
        preferred_element_type=jnp.float32)
    for e in range(n_experts):
        pltpu.make_async_copy(w1_hbm.at[e], w1s.at[e], sem1.at[e]).wait()
        w1 = w1s[e].astype(jnp.bfloat16)
        h = jax.lax.dot_general(xb, w1, (((1,), (1,)), ((), ())),
                                preferred_element_type=jnp.float32)
        h = h + b1s[e][None, :]
        h = 0.5 * h * (1.0 + jax.lax.erf(h * 0.7071067811865476))
        hs[e % 2] = (h * gates[:, e:e + 1]).astype(jnp.bfloat16)
        pltpu.make_async_copy(w2_hbm.at[e], w2s.at[e], sem2.at[e]).wait()
        w2 = w2s[e].astype(jnp.bfloat16)
        o_ref[...] += jax.lax.dot_general(
            hs[e % 2], w2, (((1,), (1,)), ((), ())),
            preferred_element_type=jnp.float32)


@jax.jit
def kernel(x, router_w, expert_w1, expert_b1, expert_w2, expert_b2):
    b, n, c = x.shape
    L = b * n
    E, dff, _ = expert_w1.shape
    x2 = x.reshape(L, c)

    out = pl.pallas_call(
        functools.partial(_moe_kernel, n_experts=E),
        in_specs=[pl.BlockSpec(memory_space=pl.ANY)] * 6,
        out_specs=pl.BlockSpec((L, c), lambda: (0, 0)),
        out_shape=jax.ShapeDtypeStruct((L, c), jnp.float32),
        scratch_shapes=[
            pltpu.VMEM((L, c), jnp.float32),
            pltpu.VMEM((E, c), jnp.float32),
            pltpu.VMEM((E, dff), jnp.float32),
            pltpu.VMEM((E, c), jnp.float32),
            pltpu.VMEM((E, dff, c), jnp.float32),
            pltpu.VMEM((E, c, dff), jnp.float32),
            pltpu.VMEM((2, L, dff), jnp.bfloat16),
            pltpu.SemaphoreType.DMA((4,)),
            pltpu.SemaphoreType.DMA((E,)),
            pltpu.SemaphoreType.DMA((E,)),
        ],
    )(x2, router_w, expert_w1, expert_b1, expert_w2, expert_b2)

    return out.reshape(b, n, c)


# speedup vs baseline: 1.4498x; 1.4498x over previous
import functools

import jax
import jax.numpy as jnp
from jax.experimental import pallas as pl
from jax.experimental.pallas import tpu as pltpu


def _moe_kernel(x_ref, rw_ref, w1_hbm, b1_ref, w2_hbm, b2_ref, o_ref,
                w1s, w2s, hs, sem1, sem2, *, n_experts):
    for e in range(n_experts):
        pltpu.make_async_copy(w1_hbm.at[e], w1s.at[e], sem1.at[e]).start()
        pltpu.make_async_copy(w2_hbm.at[e], w2s.at[e], sem2.at[e]).start()

    x = x_ref[...]
    xb = x.astype(jnp.bfloat16)

    logits = jax.lax.dot_general(
        x, rw_ref[...], (((1,), (1,)), ((), ())),
        preferred_element_type=jnp.float32)
    m = jnp.max(logits, axis=1, keepdims=True)
    ex = jnp.exp(logits - m)
    probs = ex / jnp.sum(ex, axis=1, keepdims=True)

    L = probs.shape[0]
    col = jax.lax.broadcasted_iota(jnp.int32, (L, n_experts), 1)
    m1 = jnp.max(probs, axis=1, keepdims=True)
    eq1 = probs >= m1
    i1 = jnp.min(jnp.where(eq1, col, n_experts), axis=1, keepdims=True)
    mask1 = col == i1
    rest = jnp.where(mask1, -jnp.inf, probs)
    m2 = jnp.max(rest, axis=1, keepdims=True)
    eq2 = rest >= m2
    i2 = jnp.min(jnp.where(eq2, col, n_experts), axis=1, keepdims=True)
    mask2 = col == i2

    denom = m1 + m2 + 1e-9
    gates = (jnp.where(mask1, m1, 0.0) +
             jnp.where(mask2, m2, 0.0)) / denom

    o_ref[...] = jax.lax.dot_general(
        gates, b2_ref[...], (((1,), (0,)), ((), ())),
        preferred_element_type=jnp.float32)
    for e in range(n_experts):
        pltpu.make_async_copy(w1_hbm.at[e], w1s.at[e], sem1.at[e]).wait()
        w1 = w1s[e].astype(jnp.bfloat16)
        h = jax.lax.dot_general(xb, w1, (((1,), (1,)), ((), ())),
                                preferred_element_type=jnp.float32)
        h = h + b1_ref[e][None, :]
        h = 0.5 * h * (1.0 + jax.lax.erf(h * 0.7071067811865476))
        hs[...] = (h * gates[:, e:e + 1]).astype(jnp.bfloat16)
        pltpu.make_async_copy(w2_hbm.at[e], w2s.at[e], sem2.at[e]).wait()
        w2 = w2s[e].astype(jnp.bfloat16)
        o_ref[...] += jax.lax.dot_general(
            hs[...], w2, (((1,), (1,)), ((), ())),
            preferred_element_type=jnp.float32)


@jax.jit
def kernel(x, router_w, expert_w1, expert_b1, expert_w2, expert_b2):
    b, n, c = x.shape
    L = b * n
    E, dff, _ = expert_w1.shape
    x2 = x.reshape(L, c)

    out = pl.pallas_call(
        functools.partial(_moe_kernel, n_experts=E),
        in_specs=[
            pl.BlockSpec((L, c), lambda: (0, 0)),
            pl.BlockSpec((E, c), lambda: (0, 0)),
            pl.BlockSpec(memory_space=pl.ANY),
            pl.BlockSpec((E, dff), lambda: (0, 0)),
            pl.BlockSpec(memory_space=pl.ANY),
            pl.BlockSpec((E, c), lambda: (0, 0)),
        ],
        out_specs=pl.BlockSpec((L, c), lambda: (0, 0)),
        out_shape=jax.ShapeDtypeStruct((L, c), jnp.float32),
        scratch_shapes=[
            pltpu.VMEM((E, dff, c), jnp.float32),
            pltpu.VMEM((E, c, dff), jnp.float32),
            pltpu.VMEM((L, dff), jnp.bfloat16),
            pltpu.SemaphoreType.DMA((E,)),
            pltpu.SemaphoreType.DMA((E,)),
        ],
    )(x2, router_w, expert_w1, expert_b1, expert_w2, expert_b2)

    return out.reshape(b, n, c)
